# baseline (speedup 1.0000x reference)
import functools

import jax
import jax.numpy as jnp
import numpy as np
from jax import lax
from jax.experimental import pallas as pl
from jax.experimental.pallas import tpu as pltpu
from jax.experimental.pallas import tpu_sc as plsc

N_EMBD = 128
HALF = N_EMBD // 2
L = 16
NC = 2
NS = 16
NW = NC * NS


def _angle_tables(seq_len, p_per_w):
    inv_freq = 1.0 / (10000.0 ** (np.arange(0, N_EMBD, 2, dtype=np.float32) / N_EMBD))
    coarse = (np.arange(NW, dtype=np.float32) * p_per_w)[:, None] * inv_freq[None, :]
    fine = np.arange(p_per_w, dtype=np.float32)[:, None] * inv_freq[None, :]
    a = np.concatenate([np.cos(coarse), np.sin(coarse)], axis=1)
    b = np.concatenate([np.cos(fine), np.sin(fine)], axis=1)
    return jnp.asarray(np.concatenate([a.reshape(-1), b.reshape(-1)]))


def _make_sc_kernel(batch, seq_len):
    p_per_w = seq_len // NW
    nh = HALF // L

    mesh = plsc.VectorSubcoreMesh(
        core_axis_name="c", subcore_axis_name="s", num_cores=NC, num_subcores=NS
    )

    @functools.partial(
        pl.kernel,
        out_type=jax.ShapeDtypeStruct((batch, seq_len, N_EMBD), jnp.float32),
        mesh=mesh,
        scratch_types=[
            pltpu.VMEM((batch * p_per_w,), jnp.int32),
            pltpu.VMEM((batch, p_per_w, N_EMBD), jnp.float32),
            pltpu.VMEM((p_per_w, N_EMBD), jnp.float32),
            pltpu.VMEM((N_EMBD,), jnp.float32),
            pltpu.VMEM((p_per_w * N_EMBD,), jnp.float32),
            pltpu.SemaphoreType.DMA,
            pltpu.SemaphoreType.DMA,
            pltpu.SemaphoreType.DMA,
            pltpu.SemaphoreType.DMA,
        ],
    )
    def sc_kernel(tok_hbm, w_hbm, ab_hbm, out_hbm,
                  idx_v, rows_v, cs_v, a_v, b_v, sem_i, sem_t, sem_g, sem_w):
        wid = lax.axis_index("s") * NC + lax.axis_index("c")
        pbase = wid * p_per_w

        idx_copies = [
            pltpu.async_copy(
                tok_hbm.at[b, pl.ds(pbase, p_per_w)],
                idx_v.at[pl.ds(b * p_per_w, p_per_w)],
                sem_i,
            )
            for b in range(batch)
        ]
        ta = pltpu.async_copy(ab_hbm.at[pl.ds(wid * N_EMBD, N_EMBD)], a_v, sem_t)
        tb = pltpu.async_copy(
            ab_hbm.at[pl.ds(NW * N_EMBD, p_per_w * N_EMBD)], b_v, sem_t
        )

        with jax.named_scope("idx_wait"):
            for c in idx_copies:
                c.wait()
        gathers = [
            pltpu.async_copy(
                w_hbm.at[idx_v.at[pl.ds(b * p_per_w, p_per_w)]],
                rows_v.at[b],
                sem_g,
            )
            for b in range(batch)
        ]
        with jax.named_scope("tab_wait"):
            ta.wait()
            tb.wait()

        ca = [a_v[pl.ds(j * L, L)] for j in range(nh)]
        sa = [a_v[pl.ds(HALF + j * L, L)] for j in range(nh)]

        def build(p, carry):
            pb = p * N_EMBD
            for j in range(nh):
                cb = b_v[pl.ds(pb + j * L, L)]
                sb = b_v[pl.ds(pb + HALF + j * L, L)]
                cs_v[p, pl.ds(j * L, L)] = ca[j] * cb - sa[j] * sb
                cs_v[p, pl.ds(HALF + j * L, L)] = sa[j] * cb + ca[j] * sb
            return carry

        with jax.named_scope("build"):
            lax.fori_loop(0, p_per_w, build, 0)

        writes = []
        for b in range(batch):
            with jax.named_scope(f"gwait{b}"):
                gathers[b].wait()

            def rotate(p, carry, b=b):
                ts = [rows_v[b, p, pl.ds(j * L, L)] for j in range(N_EMBD // L)]
                cs = [cs_v[p, pl.ds(j * L, L)] for j in range(N_EMBD // L)]
                for j in range(nh):
                    rows_v[b, p, pl.ds(j * L, L)] = (
                        ts[j] * cs[j] - ts[j + nh] * cs[j + nh]
                    )
                    rows_v[b, p, pl.ds((j + nh) * L, L)] = (
                        ts[j + nh] * cs[j] + ts[j] * cs[j + nh]
                    )
                return carry

            with jax.named_scope(f"rot{b}"):
                lax.fori_loop(0, p_per_w, rotate, 0)
            writes.append(
                pltpu.async_copy(
                    rows_v.at[b], out_hbm.at[b, pl.ds(pbase, p_per_w)], sem_w
                )
            )
        with jax.named_scope("wdrain"):
            for w in writes:
                w.wait()

    return sc_kernel


def kernel(token, W):
    batch, seq_len = token.shape
    ab = _angle_tables(seq_len, seq_len // NW)
    sc = _make_sc_kernel(batch, seq_len)
    return sc(token, W, ab)


# speedup vs baseline: 1.0692x; 1.0692x over previous
import functools

import jax
import jax.numpy as jnp
import numpy as np
from jax import lax
from jax.experimental import pallas as pl
from jax.experimental.pallas import tpu as pltpu
from jax.experimental.pallas import tpu_sc as plsc

N_EMBD = 128
HALF = N_EMBD // 2
L = 16
NC = 2
NS = 16
NW = NC * NS


def _angle_tables(p_per_w):
    inv_freq = 1.0 / (10000.0 ** (np.arange(0, N_EMBD, 2, dtype=np.float32) / N_EMBD))
    coarse = (np.arange(NW, dtype=np.float32) * p_per_w)[:, None] * inv_freq[None, :]
    seeds = np.concatenate([np.cos(coarse), np.sin(coarse)], axis=1)
    step = np.concatenate([np.cos(inv_freq), np.sin(inv_freq)])
    return jnp.asarray(np.concatenate([seeds.reshape(-1), step]))


def _make_sc_kernel(batch, seq_len):
    p_per_w = seq_len // NW
    nh = HALF // L

    mesh = plsc.VectorSubcoreMesh(
        core_axis_name="c", subcore_axis_name="s", num_cores=NC, num_subcores=NS
    )

    @functools.partial(
        pl.kernel,
        out_type=jax.ShapeDtypeStruct((batch, seq_len, N_EMBD), jnp.float32),
        mesh=mesh,
        scratch_types=[
            pltpu.VMEM((batch * p_per_w,), jnp.int32),
            pltpu.VMEM((batch, p_per_w, N_EMBD), jnp.float32),
            pltpu.VMEM((p_per_w, N_EMBD), jnp.float32),
            pltpu.VMEM((N_EMBD,), jnp.float32),
            pltpu.VMEM((N_EMBD,), jnp.float32),
            pltpu.SemaphoreType.DMA,
            pltpu.SemaphoreType.DMA,
            pltpu.SemaphoreType.DMA,
            pltpu.SemaphoreType.DMA,
        ],
    )
    def sc_kernel(tok_hbm, w_hbm, ab_hbm, out_hbm,
                  idx_v, rows_v, cs_v, a_v, b_v, sem_i, sem_t, sem_g, sem_w):
        wid = lax.axis_index("s") * NC + lax.axis_index("c")
        pbase = wid * p_per_w

        ta = pltpu.async_copy(ab_hbm.at[pl.ds(wid * N_EMBD, N_EMBD)], a_v, sem_t)
        tb = pltpu.async_copy(ab_hbm.at[pl.ds(NW * N_EMBD, N_EMBD)], b_v, sem_t)

        idx_copies = [
            pltpu.async_copy(
                tok_hbm.at[b, pl.ds(pbase, p_per_w)],
                idx_v.at[pl.ds(b * p_per_w, p_per_w)],
                sem_i,
            )
            for b in range(batch)
        ]
        for c in idx_copies:
            c.wait()

        gathers = [
            pltpu.async_copy(
                w_hbm.at[idx_v.at[pl.ds(b * p_per_w, p_per_w)]],
                rows_v.at[b],
                sem_g,
            )
            for b in range(batch)
        ]
        ta.wait()
        tb.wait()

        ca = [a_v[pl.ds(j * L, L)] for j in range(nh)]
        sa = [a_v[pl.ds(HALF + j * L, L)] for j in range(nh)]
        cb = [b_v[pl.ds(j * L, L)] for j in range(nh)]
        sb = [b_v[pl.ds(HALF + j * L, L)] for j in range(nh)]
        for j in range(nh):
            cs_v[0, pl.ds(j * L, L)] = ca[j]
            cs_v[0, pl.ds(HALF + j * L, L)] = sa[j]

        def build(p, carry):
            c, s = carry[:nh], carry[nh:]
            ncs = [c[j] * cb[j] - s[j] * sb[j] for j in range(nh)]
            nss = [s[j] * cb[j] + c[j] * sb[j] for j in range(nh)]
            for j in range(nh):
                cs_v[p, pl.ds(j * L, L)] = ncs[j]
                cs_v[p, pl.ds(HALF + j * L, L)] = nss[j]
            return tuple(ncs) + tuple(nss)

        lax.fori_loop(1, p_per_w, build, tuple(ca) + tuple(sa))

        writes = []
        for b in range(batch):
            gathers[b].wait()

            def rotate(p, carry, b=b):
                ts = [rows_v[b, p, pl.ds(j * L, L)] for j in range(N_EMBD // L)]
                cs = [cs_v[p, pl.ds(j * L, L)] for j in range(N_EMBD // L)]
                for j in range(nh):
                    rows_v[b, p, pl.ds(j * L, L)] = (
                        ts[j] * cs[j] - ts[j + nh] * cs[j + nh]
                    )
                    rows_v[b, p, pl.ds((j + nh) * L, L)] = (
                        ts[j + nh] * cs[j] + ts[j] * cs[j + nh]
                    )
                return carry

            lax.fori_loop(0, p_per_w, rotate, 0)
            writes.append(
                pltpu.async_copy(
                    rows_v.at[b], out_hbm.at[b, pl.ds(pbase, p_per_w)], sem_w
                )
            )
        for w in writes:
            w.wait()

    return sc_kernel


def kernel(token, W):
    batch, seq_len = token.shape
    ab = _angle_tables(seq_len // NW)
    sc = _make_sc_kernel(batch, seq_len)
    return sc(token, W, ab)


# speedup vs baseline: 1.0743x; 1.0047x over previous
import functools

import jax
import jax.numpy as jnp
import numpy as np
from jax import lax
from jax.experimental import pallas as pl
from jax.experimental.pallas import tpu as pltpu
from jax.experimental.pallas import tpu_sc as plsc

N_EMBD = 128
HALF = N_EMBD // 2
L = 16
NC = 2
NS = 16
NW = NC * NS


def _make_sc_kernel(batch, seq_len):
    p_per_w = seq_len // NW
    nh = HALF // L

    n_doubles = int(np.log2(p_per_w))
    assert (1 << n_doubles) == p_per_w

    mesh = plsc.VectorSubcoreMesh(
        core_axis_name="c", subcore_axis_name="s", num_cores=NC, num_subcores=NS
    )

    @functools.partial(
        pl.kernel,
        out_type=jax.ShapeDtypeStruct((batch, seq_len, N_EMBD), jnp.float32),
        mesh=mesh,
        scratch_types=[
            pltpu.VMEM((batch * p_per_w,), jnp.int32),
            pltpu.VMEM((batch, p_per_w, N_EMBD), jnp.float32),
            pltpu.VMEM((p_per_w, N_EMBD), jnp.float32),
            pltpu.SemaphoreType.DMA,
            pltpu.SemaphoreType.DMA,
            pltpu.SemaphoreType.DMA,
        ],
    )
    def sc_kernel(tok_hbm, w_hbm, out_hbm,
                  idx_v, rows_v, cs_v, sem_i, sem_g, sem_w):
        wid = lax.axis_index("s") * NC + lax.axis_index("c")
        pbase = wid * p_per_w

        idx_copies = [
            pltpu.async_copy(
                tok_hbm.at[b, pl.ds(pbase, p_per_w)],
                idx_v.at[pl.ds(b * p_per_w, p_per_w)],
                sem_i,
            )
            for b in range(batch)
        ]
        for c in idx_copies:
            c.wait()

        gathers = [
            pltpu.async_copy(
                w_hbm.at[idx_v.at[pl.ds(b * p_per_w, p_per_w)]],
                rows_v.at[b],
                sem_g,
            )
            for b in range(batch)
        ]

        lanes = lax.iota(jnp.int32, L).astype(jnp.float32)
        neg_scale = -float(np.log(10000.0)) / HALF
        cf, sf = [], []
        for j in range(nh):
            f = jnp.exp((lanes + float(j * L)) * neg_scale)
            x2 = f * f
            cf.append(
                1.0 + x2 * (-1.0 / 2 + x2 * (1.0 / 24 + x2 * (-1.0 / 720 + x2 * (1.0 / 40320))))
            )
            sf.append(
                f * (1.0 + x2 * (-1.0 / 6 + x2 * (1.0 / 120 + x2 * (-1.0 / 5040 + x2 * (1.0 / 362880)))))
            )
        cc, sc_ = list(cf), list(sf)
        for _ in range(n_doubles):
            cc, sc_ = (
                [cc[j] * cc[j] - sc_[j] * sc_[j] for j in range(nh)],
                [2.0 * cc[j] * sc_[j] for j in range(nh)],
            )

        def rot_step(carry, cd, sd):
            c, s = carry[:nh], carry[nh:]
            nc = [c[j] * cd[j] - s[j] * sd[j] for j in range(nh)]
            ns = [s[j] * cd[j] + c[j] * sd[j] for j in range(nh)]
            return tuple(nc) + tuple(ns)

        ones = lanes * 0.0 + 1.0
        zeros = lanes * 0.0
        seed = lax.fori_loop(
            0, wid, lambda i, cr: rot_step(cr, cc, sc_),
            (ones,) * nh + (zeros,) * nh,
        )
        for j in range(nh):
            cs_v[0, pl.ds(j * L, L)] = seed[j]
            cs_v[0, pl.ds(HALF + j * L, L)] = seed[nh + j]

        def build(p, carry):
            nxt = rot_step(carry, cf, sf)
            for j in range(nh):
                cs_v[p, pl.ds(j * L, L)] = nxt[j]
                cs_v[p, pl.ds(HALF + j * L, L)] = nxt[nh + j]
            return nxt

        lax.fori_loop(1, p_per_w, build, seed)

        writes = []
        for b in range(batch):
            gathers[b].wait()

            def rotate(p, carry, b=b):
                ts = [rows_v[b, p, pl.ds(j * L, L)] for j in range(N_EMBD // L)]
                cs = [cs_v[p, pl.ds(j * L, L)] for j in range(N_EMBD // L)]
                for j in range(nh):
                    rows_v[b, p, pl.ds(j * L, L)] = (
                        ts[j] * cs[j] - ts[j + nh] * cs[j + nh]
                    )
                    rows_v[b, p, pl.ds((j + nh) * L, L)] = (
                        ts[j + nh] * cs[j] + ts[j] * cs[j + nh]
                    )
                return carry

            lax.fori_loop(0, p_per_w, rotate, 0)
            writes.append(
                pltpu.async_copy(
                    rows_v.at[b], out_hbm.at[b, pl.ds(pbase, p_per_w)], sem_w
                )
            )
        for w in writes:
            w.wait()

    return sc_kernel


def kernel(token, W):
    batch, seq_len = token.shape
    sc = _make_sc_kernel(batch, seq_len)
    return sc(token, W)
